# baseline (speedup 1.0000x reference)
import functools

import jax
import jax.numpy as jnp
from jax import lax
from jax.experimental import pallas as pl
from jax.experimental.pallas import tpu as pltpu
from jax.experimental.pallas import tpu_sc as plsc

_NC = 2
_NS = 16
_NW = _NC * _NS
_WC = 3840
_NCH = 26
_REM0 = _NCH * _WC
_TAIL0 = _REM0 + 128


def _sc_partials(pred_hbm, tail_hbm, out_hbm,
                 b0, b1, rembuf, tailbuf, obuf, a1r, a2r, sems, semr, semt,
                 *, B, C):
    rpw = B // _NW
    ngrp = rpw // 8
    tch = ngrp * _NCH
    wid = lax.axis_index("s") * _NC + lax.axis_index("c")
    row0 = wid * rpw

    bufs = (b0, b1)
    neg = jnp.full((16,), -jnp.inf, jnp.float32)

    def start_main(t, b):
        gr0 = row0 + (t // _NCH) * 8
        col = (t % _NCH) * _WC
        pltpu.make_async_copy(
            pred_hbm.at[pl.ds(gr0, 8), pl.ds(col, _WC)], bufs[b], sems.at[b]
        ).start()

    start_main(0, 0)
    start_main(1, 1)

    def top2_fold(buf, r, a1, a2, ngroups):
        for h in range(ngroups):
            v = buf[r, pl.ds(h * 16, 16)]
            a2 = jnp.maximum(a2, jnp.minimum(a1, v))
            a1 = jnp.maximum(a1, v)
        return a1, a2

    @pl.loop(0, tch, step=2)
    def _chunks(g):
        for b in range(2):
            t = g + b
            buf = bufs[b]
            pltpu.make_async_copy(
                pred_hbm.at[pl.ds(row0, 8), pl.ds(0, _WC)], buf, sems.at[b]
            ).wait()
            rg = t // _NCH
            pos = t - rg * _NCH
            gr0 = row0 + rg * 8

            @pl.when(pos == 0)
            def _prime_group():
                pltpu.make_async_copy(
                    pred_hbm.at[pl.ds(gr0, 8), pl.ds(_REM0, 128)],
                    rembuf, semr).start()
                pltpu.make_async_copy(
                    tail_hbm.at[pl.ds(gr0, 8)], tailbuf, semt).start()
                for i in range(8):
                    a1r[pl.ds(16 * i, 16)] = neg
                    a2r[pl.ds(16 * i, 16)] = neg

            carry = []
            for r in range(8):
                carry += [a1r[pl.ds(16 * r, 16)], a2r[pl.ds(16 * r, 16)]]
            carry = tuple(carry)

            def mb(tt, carry):
                out = []
                for r in range(8):
                    a1, a2 = carry[2 * r], carry[2 * r + 1]
                    for h in range(8):
                        v = buf[r, pl.ds(tt * 128 + h * 16, 16)]
                        a2 = jnp.maximum(a2, jnp.minimum(a1, v))
                        a1 = jnp.maximum(a1, v)
                    out += [a1, a2]
                return tuple(out)

            carry = lax.fori_loop(0, _WC // 128, mb, carry)
            for r in range(8):
                a1r[pl.ds(16 * r, 16)] = carry[2 * r]
                a2r[pl.ds(16 * r, 16)] = carry[2 * r + 1]

            @pl.when(t + 2 < tch)
            def _prefetch():
                start_main(t + 2, b)

            @pl.when(pos == _NCH - 1)
            def _finalize():
                pltpu.make_async_copy(
                    pred_hbm.at[pl.ds(row0, 8), pl.ds(0, 128)],
                    rembuf, semr).wait()
                pltpu.make_async_copy(
                    pred_hbm.at[pl.ds(row0, 8), pl.ds(0, 128)],
                    tailbuf, semt).wait()
                for r in range(8):
                    rl = rg * 8 + r
                    a1, a2 = a1r[pl.ds(16 * r, 16)], a2r[pl.ds(16 * r, 16)]
                    a1, a2 = top2_fold(rembuf, r, a1, a2, 8)
                    a1, a2 = top2_fold(tailbuf, r, a1, a2, 8)
                    obuf[pl.ds(rl * 32, 16)] = a1
                    obuf[pl.ds(rl * 32 + 16, 16)] = a2

    pltpu.sync_copy(obuf, out_hbm.at[pl.ds(row0 * 32, rpw * 32)])


def _tc_combine(k_smem, part_ref, k2d_ref, pred_hbm, out_ref, bufs, sems):
    i = pl.program_id(0)
    ni = pl.num_programs(0)

    def start_row_copies(step, s):
        base = step * 8
        for r in range(8):
            c0 = (k_smem[base + r] // 128) * 128
            pltpu.make_async_copy(
                pred_hbm.at[pl.ds(base, 8), pl.ds(c0, 128)],
                bufs.at[s * 8 + r], sems.at[s * 8 + r]).start()

    @pl.when(i == 0)
    def _prologue():
        start_row_copies(0, 0)

    @pl.when(i + 1 < ni)
    def _prefetch():
        start_row_copies(i + 1, (i + 1) % 2)

    a1 = part_ref[:, :16]
    a2 = part_ref[:, 16:]
    lk = jax.lax.rem(k2d_ref[...], 16)
    oh = jax.lax.broadcasted_iota(jnp.int32, (8, 16), 1) == lk
    ninf = jnp.float32(-jnp.inf)
    m_wo = jnp.where(oh, ninf, a1).max(axis=1)
    a1k = jnp.where(oh, a1, ninf).max(axis=1)
    a2k = jnp.where(oh, a2, ninf).max(axis=1)

    s = i % 2
    lane128 = jax.lax.broadcasted_iota(jnp.int32, (1, 128), 1)
    pks = []
    for r in range(8):
        c0 = (k_smem[i * 8 + r] // 128) * 128
        pltpu.make_async_copy(
            pred_hbm.at[pl.ds(i * 8, 8), pl.ds(0, 128)],
            bufs.at[s * 8 + r], sems.at[s * 8 + r]).wait()
        row = bufs[s * 8 + r, pl.ds(r, 1), :]
        ohc = lane128 == (k_smem[i * 8 + r] - c0)
        pks.append(jnp.where(ohc, row, ninf).max(axis=1))
    pk = jnp.concatenate(pks)

    cls = jnp.where(a1k == pk, a2k, a1k)
    out_ref[...] = (jnp.maximum(m_wo, cls) - pk)[:, None]


def kernel(prediction, k):
    B, C = prediction.shape
    k2 = k.astype(jnp.int32)
    tail = jnp.pad(prediction[:, _TAIL0:], ((0, 0), (0, 128 - (C - _TAIL0))),
                   constant_values=-jnp.inf)
    rpw = B // _NW
    mesh = plsc.VectorSubcoreMesh(core_axis_name="c", subcore_axis_name="s")
    partials = pl.kernel(
        functools.partial(_sc_partials, B=B, C=C),
        out_type=jax.ShapeDtypeStruct((B * 32,), jnp.float32),
        mesh=mesh,
        scratch_types=[
            pltpu.VMEM((8, _WC), jnp.float32),
            pltpu.VMEM((8, _WC), jnp.float32),
            pltpu.VMEM((8, 128), jnp.float32),
            pltpu.VMEM((8, 128), jnp.float32),
            pltpu.VMEM((rpw * 32,), jnp.float32),
            pltpu.VMEM((128,), jnp.float32),
            pltpu.VMEM((128,), jnp.float32),
            pltpu.SemaphoreType.DMA((2,)),
            pltpu.SemaphoreType.DMA,
            pltpu.SemaphoreType.DMA,
        ],
        compiler_params=pltpu.CompilerParams(use_tc_tiling_on_sc=True),
    )(prediction, tail)

    part2d = partials.reshape(B, 32)
    out = pl.pallas_call(
        _tc_combine,
        grid=(B // 8,),
        in_specs=[
            pl.BlockSpec(memory_space=pltpu.SMEM),
            pl.BlockSpec((8, 32), lambda i: (i, 0)),
            pl.BlockSpec((8, 1), lambda i: (i, 0)),
            pl.BlockSpec(memory_space=pltpu.MemorySpace.HBM),
        ],
        out_specs=pl.BlockSpec((8, 1), lambda i: (i, 0)),
        out_shape=jax.ShapeDtypeStruct((B, 1), jnp.float32),
        scratch_shapes=[
            pltpu.VMEM((16, 8, 128), jnp.float32),
            pltpu.SemaphoreType.DMA((16,)),
        ],
        compiler_params=pltpu.CompilerParams(
            dimension_semantics=("arbitrary",),
        ),
    )(k2, part2d, k2.reshape(B, 1), prediction)
    return out.reshape(B)


# speedup vs baseline: 1.1994x; 1.1994x over previous
import functools

import jax
import jax.numpy as jnp
from jax.experimental import pallas as pl
from jax.experimental.pallas import tpu as pltpu

_R = 8
_NBUF = 8


def _margin_kernel(k_ref, pred_hbm, out_ref, bufs, tails, pk_acc, sems, semt,
                   *, B, C):
    C_al = (C // 128) * 128
    tw = C - C_al
    nchunks = B // _R
    lane = jax.lax.broadcasted_iota(jnp.int32, (1, 128), 1)
    tlane = jax.lax.broadcasted_iota(jnp.int32, (1, tw), 1)

    def start_copy(t, b, prio):
        rows = pl.ds(t * _R, _R)
        pltpu.async_copy(
            pred_hbm.at[rows, pl.ds(0, C_al)], bufs.at[b], sems.at[b],
            priority=prio)
        pltpu.async_copy(
            pred_hbm.at[rows, pl.ds(C_al, tw)], tails.at[b], semt.at[b],
            priority=prio)

    for t in range(_NBUF):
        start_copy(t, t, t % 2)

    def body(t, carry):
        b = jax.lax.rem(t, _NBUF)
        rows = pl.ds(t * _R, _R)
        pltpu.make_async_copy(
            pred_hbm.at[rows, pl.ds(0, C_al)], bufs.at[b], sems.at[b]).wait()
        pltpu.make_async_copy(
            pred_hbm.at[rows, pl.ds(C_al, tw)], tails.at[b], semt.at[b]).wait()

        for r in range(_R):
            c = k_ref[t * _R + r]

            def _bulk_rmw(c=c, r=r, b=b):
                c0 = (c // 128) * 128
                chunk = bufs[b, pl.ds(r, 1), pl.ds(c0, 128)]
                is_l = lane == (c - c0)
                pk_acc[pl.ds(r, 1), :] = jnp.where(is_l, chunk, -jnp.inf).max(
                    axis=1, keepdims=True)
                bufs[b, pl.ds(r, 1), pl.ds(c0, 128)] = jnp.where(
                    is_l, -jnp.inf, chunk)

            def _tail_rmw(c=c, r=r, b=b):
                chunk = tails[b, pl.ds(r, 1), :]
                is_l = tlane == (c - C_al)
                pk_acc[pl.ds(r, 1), :] = jnp.where(is_l, chunk, -jnp.inf).max(
                    axis=1, keepdims=True)
                tails[b, pl.ds(r, 1), :] = jnp.where(is_l, -jnp.inf, chunk)

            pl.when(c < C_al)(_bulk_rmw)
            pl.when(c >= C_al)(_tail_rmw)

        m = jnp.maximum(jnp.max(bufs[b], axis=1), jnp.max(tails[b], axis=1))
        out_ref[pl.ds(t * _R, _R), :] = m[:, None] - pk_acc[...]

        nxt = t + _NBUF

        @pl.when((nxt < nchunks) & (jax.lax.rem(nxt, 2) == 0))
        def _():
            start_copy(nxt, b, 0)

        @pl.when((nxt < nchunks) & (jax.lax.rem(nxt, 2) == 1))
        def _():
            start_copy(nxt, b, 1)

        return carry

    jax.lax.fori_loop(0, nchunks, body, 0, unroll=False)


def kernel(prediction, k):
    B, C = prediction.shape
    k2 = k.astype(jnp.int32)
    C_al = (C // 128) * 128
    tw = C - C_al
    out = pl.pallas_call(
        functools.partial(_margin_kernel, B=B, C=C),
        in_specs=[
            pl.BlockSpec(memory_space=pltpu.SMEM),
            pl.BlockSpec(memory_space=pltpu.MemorySpace.HBM),
        ],
        out_specs=pl.BlockSpec(memory_space=pltpu.VMEM),
        out_shape=jax.ShapeDtypeStruct((B, 1), jnp.float32),
        scratch_shapes=[
            pltpu.VMEM((_NBUF, _R, C_al), jnp.float32),
            pltpu.VMEM((_NBUF, _R, tw), jnp.float32),
            pltpu.VMEM((_R, 1), jnp.float32),
            pltpu.SemaphoreType.DMA((_NBUF,)),
            pltpu.SemaphoreType.DMA((_NBUF,)),
        ],
    )(k2, prediction)
    return out.reshape(B)
